# baseline (speedup 1.0000x reference)
import functools

import jax
import jax.numpy as jnp
from jax import lax
from jax.experimental import pallas as pl
from jax.experimental.pallas import tpu as pltpu
from jax.experimental.pallas import tpu_sc as plsc

_OUT_LEN = 4096
_IN_LEN = 8192
_B = 32
_C = 32
_NPAIR = _B * _C

_WIN = 4112
_WIN_PAD = 4224
_S0MAX = _IN_LEN - _WIN

_INFO = plsc.get_sparse_core_info()
_NC = _INFO.num_cores
_NS = _INFO.num_subcores
_NW = _NC * _NS
_ROWS_PER_W = _NPAIR // _NW


@functools.partial(
    pl.kernel,
    out_type=jax.ShapeDtypeStruct((_NPAIR, _OUT_LEN), jnp.float32),
    mesh=plsc.VectorSubcoreMesh(core_axis_name="c", subcore_axis_name="s"),
    scratch_types=[
        pltpu.VMEM((_WIN_PAD,), jnp.float32),
        pltpu.VMEM((_WIN_PAD,), jnp.float32),
        pltpu.VMEM((_OUT_LEN,), jnp.float32),
        pltpu.VMEM((_OUT_LEN,), jnp.float32),
        pltpu.VMEM((_OUT_LEN,), jnp.float32),
        pltpu.VMEM((128,), jnp.float32),
        pltpu.SemaphoreType.DMA,
        pltpu.SemaphoreType.DMA,
        pltpu.SemaphoreType.DMA,
        pltpu.SemaphoreType.DMA,
    ],
    compiler_params=pltpu.CompilerParams(
        needs_layout_passes=False, use_tc_tiling_on_sc=False
    ),
)
def _sc_interp(
    sigT_hbm, traf_hbm, lin_hbm, out_hbm,
    win0_v, win1_v, orow0_v, orow1_v, lin_v, traf_v,
    si0, si1, so0, so1,
):
    wid = lax.axis_index("s") * _NC + lax.axis_index("c")
    base = wid * _ROWS_PER_W
    pltpu.sync_copy(lin_hbm, lin_v)
    pltpu.sync_copy(
        traf_hbm.at[pl.ds(base * 2, _ROWS_PER_W * 2)],
        traf_v.at[pl.ds(0, _ROWS_PER_W * 2)],
    )

    def row_start(j):
        a1 = plsc.load_gather(traf_v, [jnp.full((16,), 2 * j + 1, jnp.int32)])
        bi = (a1 * jnp.float32(4096.0)).astype(jnp.int32)
        s0v = jnp.minimum(bi & ~7, _S0MAX)
        return pl.multiple_of(jnp.max(s0v, axis=0), 8)

    def start_in(p, j, win_v, si):
        s0 = row_start(j)
        pltpu.async_copy(
            sigT_hbm.at[p, pl.ds(s0, _WIN)], win_v.at[pl.ds(0, _WIN)], si
        )

    def wait_in(p, j, win_v, si):
        s0 = row_start(j)
        pltpu.make_async_copy(
            sigT_hbm.at[p, pl.ds(s0, _WIN)], win_v.at[pl.ds(0, _WIN)], si
        ).wait()

    def wait_out(p, orow_v, so):
        pltpu.make_async_copy(orow_v, out_hbm.at[p], so).wait()

    def do_row(p, j, win_v, orow_v, so):
        a0 = plsc.load_gather(traf_v, [jnp.full((16,), 2 * j, jnp.int32)])
        a1 = plsc.load_gather(traf_v, [jnp.full((16,), 2 * j + 1, jnp.int32)])
        scale_a = a0 * jnp.float32(4096.0)
        scale_b = a1 * jnp.float32(4096.0)
        bi = scale_b.astype(jnp.int32)
        s0f = jnp.minimum(bi & ~7, _S0MAX).astype(jnp.float32)
        limv = jnp.float32(_IN_LEN - 1) - s0f

        @plsc.parallel_loop(0, _OUT_LEN, step=16, unroll=8)
        def blk(i):
            linv = lin_v[pl.ds(i, 16)]
            xs = scale_a * linv + scale_b - s0f
            x0i = jnp.minimum(xs, limv).astype(jnp.int32)
            x0f = x0i.astype(jnp.float32)
            x1f = jnp.minimum(x0f + jnp.float32(1.0), limv)
            v0 = plsc.load_gather(win_v, [x0i])
            v1 = plsc.load_gather(win_v, [x1f.astype(jnp.int32)])
            w0 = x1f - xs
            w1 = xs - x0f
            orow_v[pl.ds(i, 16)] = w0 * v0 + w1 * v1

        pltpu.async_copy(orow_v, out_hbm.at[p], so)

    start_in(base, 0, win0_v, si0)

    def pair_body(k, carry):
        p0 = base + 2 * k
        wait_in(p0, 2 * k, win0_v, si0)
        start_in(p0 + 1, 2 * k + 1, win1_v, si1)

        @pl.when(k > 0)
        def _():
            wait_out(p0 - 2, orow0_v, so0)

        do_row(p0, 2 * k, win0_v, orow0_v, so0)

        wait_in(p0 + 1, 2 * k + 1, win1_v, si1)

        @pl.when(k < _ROWS_PER_W // 2 - 1)
        def _():
            start_in(p0 + 2, 2 * k + 2, win0_v, si0)

        @pl.when(k > 0)
        def _():
            wait_out(p0 - 1, orow1_v, so1)

        do_row(p0 + 1, 2 * k + 1, win1_v, orow1_v, so1)
        return carry

    lax.fori_loop(0, _ROWS_PER_W // 2, pair_body, 0)
    wait_out(base + _ROWS_PER_W - 2, orow0_v, so0)
    wait_out(base + _ROWS_PER_W - 1, orow1_v, so1)


def kernel(transformation, sig):
    sigT = jnp.transpose(sig, (0, 2, 1)).reshape(_NPAIR, _IN_LEN)
    traf = lax.reduce_precision(
        transformation.reshape(_NPAIR * 2), exponent_bits=8, mantissa_bits=7
    )
    lin = lax.reduce_precision(
        jnp.linspace(0.0, 1.0, _OUT_LEN), exponent_bits=8, mantissa_bits=7
    )
    outT = _sc_interp(sigT, traf, lin)
    return jnp.transpose(outT.reshape(_B, _C, _OUT_LEN), (0, 2, 1))


# speedup vs baseline: 1.9978x; 1.9978x over previous
import functools

import jax
import jax.numpy as jnp
from jax import lax
from jax.experimental import pallas as pl
from jax.experimental.pallas import tpu as pltpu
from jax.experimental.pallas import tpu_sc as plsc

_OUT_LEN = 4096
_IN_LEN = 8192
_B = 32
_C = 32
_NPAIR = _B * _C

_INFO = plsc.get_sparse_core_info()
_NC = _INFO.num_cores
_NS = _INFO.num_subcores
_NW = _NC * _NS
_ROWS_PER_W = _NPAIR // _NW
_NBUF = 4


@functools.partial(
    pl.kernel,
    out_type=jax.ShapeDtypeStruct((_NPAIR, _OUT_LEN), jnp.float32),
    mesh=plsc.VectorSubcoreMesh(core_axis_name="c", subcore_axis_name="s"),
    scratch_types=(
        [pltpu.VMEM((_IN_LEN,), jnp.float32) for _ in range(_NBUF)]
        + [pltpu.VMEM((_OUT_LEN,), jnp.float32) for _ in range(_NBUF)]
        + [
            pltpu.VMEM((_OUT_LEN,), jnp.float32),
            pltpu.VMEM((128,), jnp.float32),
        ]
        + [pltpu.SemaphoreType.DMA for _ in range(2 * _NBUF)]
    ),
    compiler_params=pltpu.CompilerParams(needs_layout_passes=False),
)
def _sc_interp(sigT_hbm, traf_hbm, lin_hbm, out_hbm, *scratch):
    rows = scratch[:_NBUF]
    orows = scratch[_NBUF:2 * _NBUF]
    lin_v = scratch[2 * _NBUF]
    traf_v = scratch[2 * _NBUF + 1]
    sis = scratch[2 * _NBUF + 2:2 * _NBUF + 2 + _NBUF]
    sos = scratch[2 * _NBUF + 2 + _NBUF:]

    wid = lax.axis_index("s") * _NC + lax.axis_index("c")
    base = wid * _ROWS_PER_W
    pltpu.sync_copy(lin_hbm, lin_v)
    pltpu.sync_copy(
        traf_hbm.at[pl.ds(base * 2, _ROWS_PER_W * 2)],
        traf_v.at[pl.ds(0, _ROWS_PER_W * 2)],
    )
    for b in range(_NBUF):
        pltpu.async_copy(sigT_hbm.at[base + b], rows[b], sis[b])

    lim = jnp.float32(_IN_LEN - 1)

    def do_row(p, j, row_v, orow_v, so):
        a0 = plsc.load_gather(traf_v, [jnp.full((16,), 2 * j, jnp.int32)])
        a1 = plsc.load_gather(traf_v, [jnp.full((16,), 2 * j + 1, jnp.int32)])
        scale_a = a0 * jnp.float32(4096.0)
        scale_b = a1 * jnp.float32(4096.0)

        @plsc.parallel_loop(0, _OUT_LEN, step=16, unroll=8)
        def blk(i):
            linv = lin_v[pl.ds(i, 16)]
            x = scale_a * linv + scale_b
            x0i = jnp.minimum(x, lim).astype(jnp.int32)
            x0f = x0i.astype(jnp.float32)
            x1f = jnp.minimum(x0f + jnp.float32(1.0), lim)
            v0 = plsc.load_gather(row_v, [x0i])
            v1 = plsc.load_gather(row_v, [x1f.astype(jnp.int32)])
            w0 = x1f - x
            w1 = x - x0f
            orow_v[pl.ds(i, 16)] = w0 * v0 + w1 * v1

        pltpu.async_copy(orow_v, out_hbm.at[p], so)

    def group_body(k, carry):
        for b in range(_NBUF):
            j = k * _NBUF + b
            p = base + j
            pltpu.make_async_copy(sigT_hbm.at[p], rows[b], sis[b]).wait()

            @pl.when(k > 0)
            def _():
                pltpu.make_async_copy(
                    orows[b], out_hbm.at[p - _NBUF], sos[b]
                ).wait()

            do_row(p, j, rows[b], orows[b], sos[b])

            @pl.when(j + _NBUF < _ROWS_PER_W)
            def _():
                pltpu.async_copy(sigT_hbm.at[p + _NBUF], rows[b], sis[b])

        return carry

    lax.fori_loop(0, _ROWS_PER_W // _NBUF, group_body, 0)
    for b in range(_NBUF):
        pltpu.make_async_copy(
            orows[b], out_hbm.at[base + _ROWS_PER_W - _NBUF + b], sos[b]
        ).wait()


def kernel(transformation, sig):
    sigT = jnp.transpose(sig, (0, 2, 1)).reshape(_NPAIR, _IN_LEN)
    traf = lax.reduce_precision(
        transformation.reshape(_NPAIR * 2), exponent_bits=8, mantissa_bits=7
    )
    lin = lax.reduce_precision(
        jnp.linspace(0.0, 1.0, _OUT_LEN), exponent_bits=8, mantissa_bits=7
    )
    outT = _sc_interp(sigT, traf, lin)
    return jnp.transpose(outT.reshape(_B, _C, _OUT_LEN), (0, 2, 1))
